# baseline (speedup 1.0000x reference)
import functools

import jax
import jax.numpy as jnp
from jax import lax
from jax.experimental import pallas as pl
from jax.experimental.pallas import tpu as pltpu
from jax.experimental.pallas import tpu_sc as plsc

ROWS = 64
COLS = 32768
NC = 2
NS = 16
NW = NC * NS
RPW = ROWS // NW
LANES = 16
ACC = 8
UNROLL = 32
CHUNK = 8192
NCHUNK = COLS // CHUNK

_mesh = plsc.VectorSubcoreMesh(core_axis_name="c", subcore_axis_name="s")


@functools.partial(
    pl.kernel,
    mesh=_mesh,
    out_type=jax.ShapeDtypeStruct((NW, LANES), jnp.int32),
    scratch_types=[
        pltpu.VMEM((RPW, COLS), jnp.float32),
        pltpu.VMEM((LANES,), jnp.int32),
    ] + [pltpu.SemaphoreType.DMA] * (RPW * NCHUNK),
)
def _argmax_sc(x_hbm, out_hbm, buf, obuf, *sems):
    wid = lax.axis_index("c") * NS + lax.axis_index("s")
    base_row = wid * RPW

    cps = []
    for r in range(RPW):
        for c in range(NCHUNK):
            cps.append(pltpu.async_copy(
                x_hbm.at[pl.ds(base_row + r, 1), pl.ds(c * CHUNK, CHUNK)],
                buf.at[pl.ds(r, 1), pl.ds(c * CHUNK, CHUNK)],
                sems[r * NCHUNK + c]))

    lane = lax.iota(jnp.int32, 16)
    res = jnp.zeros((LANES,), jnp.int32)
    for r in range(RPW):
        ms = [jnp.full((LANES,), -jnp.inf, jnp.float32)] * ACC
        bis = [jnp.zeros((LANES,), jnp.int32)] * ACC
        for c in range(NCHUNK):
            cps[r * NCHUNK + c].wait()
            cbase = c * CHUNK

            def body(jo, carry, r=r, cbase=cbase):
                ms = list(carry[0])
                bis = list(carry[1])
                base = cbase + jo * (LANES * UNROLL)
                for u in range(UNROLL):
                    a = u % ACC
                    eb = base + u * LANES
                    v = buf[r, pl.ds(eb, LANES)]
                    gt = v > ms[a]
                    ms[a] = jnp.where(gt, v, ms[a])
                    bis[a] = jnp.where(gt, eb, bis[a])
                return tuple(ms), tuple(bis)

            out_c = lax.fori_loop(0, CHUNK // (LANES * UNROLL), body,
                                  (tuple(ms), tuple(bis)))
            ms, bis = list(out_c[0]), list(out_c[1])

        stride = ACC
        while stride > 1:
            stride //= 2
            for a in range(stride):
                m2, b2 = ms[a + stride], bis[a + stride]
                tk = (m2 > ms[a]) | ((m2 == ms[a]) & (b2 < bis[a]))
                ms[a] = jnp.where(tk, m2, ms[a])
                bis[a] = jnp.where(tk, b2, bis[a])

        mv, iv = ms[0], bis[0] + lane
        for k in (8, 4, 2, 1):
            perm = lane ^ k
            pm = mv.at[perm].get(mode="promise_in_bounds")
            pi = iv.at[perm].get(mode="promise_in_bounds")
            tk = (pm > mv) | ((pm == mv) & (pi < iv))
            mv = jnp.where(tk, pm, mv)
            iv = jnp.where(tk, pi, iv)
        res = jnp.where(lane == r, iv, res)

    obuf[...] = res
    pltpu.sync_copy(obuf, out_hbm.at[wid])


def kernel(x):
    out = _argmax_sc(x)
    return out[:, :RPW].reshape(ROWS)


# speedup vs baseline: 1.1265x; 1.1265x over previous
import functools

import jax
import jax.numpy as jnp
from jax import lax
from jax.experimental import pallas as pl
from jax.experimental.pallas import tpu as pltpu
from jax.experimental.pallas import tpu_sc as plsc

ROWS = 64
COLS = 32768
NC = 2
NS = 16
NW = NC * NS
RPW = ROWS // NW
LANES = 16
ACC = 4
UNROLL = 16
CHUNK = 8192
NCHUNK = COLS // CHUNK

_mesh = plsc.VectorSubcoreMesh(core_axis_name="c", subcore_axis_name="s")


@functools.partial(
    pl.kernel,
    mesh=_mesh,
    out_type=jax.ShapeDtypeStruct((NW, LANES), jnp.int32),
    scratch_types=[
        pltpu.VMEM((RPW, COLS), jnp.float32),
        pltpu.VMEM((LANES,), jnp.int32),
    ] + [pltpu.SemaphoreType.DMA] * (RPW * NCHUNK),
)
def _argmax_sc(x_hbm, out_hbm, buf, obuf, *sems):
    wid = lax.axis_index("c") * NS + lax.axis_index("s")
    base_row = wid * RPW

    cps = []
    for r in range(RPW):
        for c in range(NCHUNK):
            cps.append(pltpu.async_copy(
                x_hbm.at[pl.ds(base_row + r, 1), pl.ds(c * CHUNK, CHUNK)],
                buf.at[pl.ds(r, 1), pl.ds(c * CHUNK, CHUNK)],
                sems[r * NCHUNK + c]))

    lane = lax.iota(jnp.int32, 16)
    res = jnp.zeros((LANES,), jnp.int32)
    for r in range(RPW):
        ms = [jnp.full((LANES,), -jnp.inf, jnp.float32)] * ACC
        bis = [jnp.zeros((LANES,), jnp.int32)] * ACC
        for c in range(NCHUNK):
            cps[r * NCHUNK + c].wait()
            cbase = c * CHUNK

            def body(jo, carry, r=r, cbase=cbase):
                ms = list(carry[0])
                bis = list(carry[1])
                base = cbase + jo * (LANES * UNROLL)
                for u in range(UNROLL):
                    a = u % ACC
                    eb = base + u * LANES
                    v = buf[r, pl.ds(eb, LANES)]
                    gt = v > ms[a]
                    ms[a] = jnp.maximum(v, ms[a])
                    bis[a] = jnp.where(gt, eb, bis[a])
                return tuple(ms), tuple(bis)

            out_c = lax.fori_loop(0, CHUNK // (LANES * UNROLL), body,
                                  (tuple(ms), tuple(bis)))
            ms, bis = list(out_c[0]), list(out_c[1])

        stride = ACC
        while stride > 1:
            stride //= 2
            for a in range(stride):
                m2, b2 = ms[a + stride], bis[a + stride]
                tk = (m2 > ms[a]) | ((m2 == ms[a]) & (b2 < bis[a]))
                ms[a] = jnp.where(tk, m2, ms[a])
                bis[a] = jnp.where(tk, b2, bis[a])

        mv, iv = ms[0], bis[0] + lane
        for k in (8, 4, 2, 1):
            perm = lane ^ k
            pm = mv.at[perm].get(mode="promise_in_bounds")
            pi = iv.at[perm].get(mode="promise_in_bounds")
            tk = (pm > mv) | ((pm == mv) & (pi < iv))
            mv = jnp.where(tk, pm, mv)
            iv = jnp.where(tk, pi, iv)
        res = jnp.where(lane == r, iv, res)

    obuf[...] = res
    pltpu.sync_copy(obuf, out_hbm.at[wid])


def kernel(x):
    out = _argmax_sc(x)
    return out[:, :RPW].reshape(ROWS)
